# baseline (speedup 1.0000x reference)
import jax
import jax.numpy as jnp
from jax.experimental import pallas as pl

NUM_EXPERT = 64
IN_FEAT = 768
OUT_FEAT = 768
N_TOKENS = 128
EPG = 4
NSTEPS = NUM_EXPERT // EPG


def _moe_kernel(gate_ref, inp_ref, w_ref, out_ref):
    s = pl.program_id(0)

    @pl.when(s == 0)
    def _init():
        out_ref[...] = jnp.zeros_like(out_ref)

    acc = out_ref[...]
    for j in range(EPG):
        e = s * EPG + j
        mask = gate_ref[...] == e
        x = jnp.where(mask, inp_ref[...], 0.0)
        acc += jax.lax.dot_general(
            x.astype(jnp.bfloat16), w_ref[j].astype(jnp.bfloat16),
            (((1,), (1,)), ((), ())),
            preferred_element_type=jnp.float32,
        )
    out_ref[...] = acc


def kernel(inp, gate, weight):
    gate2d = gate.reshape(N_TOKENS, 1)
    return pl.pallas_call(
        _moe_kernel,
        grid=(NSTEPS,),
        in_specs=[
            pl.BlockSpec((N_TOKENS, 1), lambda s: (0, 0)),
            pl.BlockSpec((N_TOKENS, IN_FEAT), lambda s: (0, 0)),
            pl.BlockSpec((EPG, OUT_FEAT, IN_FEAT), lambda s: (s, 0, 0)),
        ],
        out_specs=pl.BlockSpec((N_TOKENS, OUT_FEAT), lambda s: (0, 0)),
        out_shape=jax.ShapeDtypeStruct((N_TOKENS, OUT_FEAT), jnp.float32),
    )(gate2d, inp, weight)


# speedup vs baseline: 1.0043x; 1.0043x over previous
import jax
import jax.numpy as jnp
from jax.experimental import pallas as pl

NUM_EXPERT = 64
IN_FEAT = 768
OUT_FEAT = 768
N_TOKENS = 128
EPG = 4
NSTEPS = NUM_EXPERT // EPG


def _moe_kernel(gate_ref, inp_ref, w_ref, out_ref):
    s = pl.program_id(0)

    @pl.when(s == 0)
    def _init():
        out_ref[...] = jnp.zeros_like(out_ref)

    acc = out_ref[...]
    for j in range(EPG):
        e = s * EPG + j
        mask = gate_ref[...] == e
        x = jnp.where(mask, inp_ref[...], 0.0)
        acc += jax.lax.dot_general(
            x, w_ref[j],
            (((1,), (1,)), ((), ())),
            preferred_element_type=jnp.float32,
        )
    out_ref[...] = acc


def kernel(inp, gate, weight):
    gate2d = gate.reshape(N_TOKENS, 1)
    return pl.pallas_call(
        _moe_kernel,
        grid=(NSTEPS,),
        in_specs=[
            pl.BlockSpec((N_TOKENS, 1), lambda s: (0, 0)),
            pl.BlockSpec((N_TOKENS, IN_FEAT), lambda s: (0, 0)),
            pl.BlockSpec((EPG, OUT_FEAT, IN_FEAT), lambda s: (s, 0, 0)),
        ],
        out_specs=pl.BlockSpec((N_TOKENS, OUT_FEAT), lambda s: (0, 0)),
        out_shape=jax.ShapeDtypeStruct((N_TOKENS, OUT_FEAT), jnp.float32),
    )(gate2d, inp, weight)
